# Initial kernel scaffold; baseline (speedup 1.0000x reference)
import jax
import jax.numpy as jnp
from jax.experimental import pallas as pl


def kernel(h, edge_attr, edge_index, W_h, b_h, W_r1a, b_r1a, W_r1b, b_r1b, W_r2a, b_r2a, W_r2b, b_r2b, W_r3a, b_r3a, W_r3b, b_r3b, W_m, b_m, W_xe, b_xe, W_lin):
    raise NotImplementedError("write your pallas kernel here")


# speedup vs baseline: 3.2525x; 3.2525x over previous
import functools

import jax
import jax.numpy as jnp
from jax import lax
from jax.experimental import pallas as pl
from jax.experimental.pallas import tpu as pltpu
from jax.experimental.pallas import tpu_sc as plsc

_N = 10000
_E = 320000
_D = 128

_NC = 2
_NS = 16
_NW = _NC * _NS
_EPW = _E // _NW
_K = 80
_NCHUNK = _EPW // _K
_NP = 10240
_RPT = _NP // _NS

_F32 = jnp.float32


def _silu(y):
    return y * jax.nn.sigmoid(y)


def _dot(a, w):
    return jnp.dot(a, w, preferred_element_type=_F32)


def _edge_proj_body(ea_ref, w3_ref, b_ref, wl_ref, t3_ref, tl_ref):
    ea = ea_ref[...]
    t3_ref[...] = _dot(ea, w3_ref[...]) + b_ref[...]
    tl_ref[...] = _dot(ea, wl_ref[...])


def _edge_proj(ea, w3, b_xe, w_lin, block=2000):
    grid = (_E // block,)
    return pl.pallas_call(
        _edge_proj_body,
        grid=grid,
        in_specs=[
            pl.BlockSpec((block, _D), lambda i: (i, 0)),
            pl.BlockSpec((_D, _D), lambda i: (0, 0)),
            pl.BlockSpec((1, _D), lambda i: (0, 0)),
            pl.BlockSpec((_D, _D), lambda i: (0, 0)),
        ],
        out_specs=[
            pl.BlockSpec((block, _D), lambda i: (i, 0)),
            pl.BlockSpec((block, _D), lambda i: (i, 0)),
        ],
        out_shape=[
            jax.ShapeDtypeStruct((_E, _D), _F32),
            jax.ShapeDtypeStruct((_E, _D), _F32),
        ],
    )(ea, w3, b_xe.reshape(1, _D), w_lin)


def _pre_body(h_ref, wh_ref, bh_ref, w1_ref, w2_ref, x0_ref, p1_ref, p2_ref):
    x0 = _silu(_dot(h_ref[...], wh_ref[...]) + bh_ref[...])
    x0_ref[...] = x0
    p1_ref[...] = _dot(x0, w1_ref[...])
    p2_ref[...] = _dot(x0, w2_ref[...])


def _pre(h, w_h, b_h, w1, w2, block=1000):
    grid = (_N // block,)
    mat = pl.BlockSpec((_D, _D), lambda i: (0, 0))
    row = pl.BlockSpec((block, _D), lambda i: (i, 0))
    return pl.pallas_call(
        _pre_body,
        grid=grid,
        in_specs=[row, mat, pl.BlockSpec((1, _D), lambda i: (0, 0)), mat, mat],
        out_specs=[row, row, row],
        out_shape=[jax.ShapeDtypeStruct((_N, _D), _F32)] * 3,
    )(h, w_h, b_h.reshape(1, _D), w1, w2)


def _mid_body(s_ref, c_ref, x0_ref, h_ref,
              w1a, b1a, w1b, b1b, wm, bm,
              w2a, b2a, w2b, b2b, w3a, b3a, w3b, b3b,
              wxe1, wxe2,
              x6_ref, p1_ref, p2_ref):
    cnt = jnp.maximum(c_ref[0, :, :1] + c_ref[1, :, :1], 1.0)
    pooled = (s_ref[0] + s_ref[1]) / cnt
    x1 = pooled + x0_ref[...]
    y = _silu(_dot(x1, w1a[...]) + b1a[...])
    y = _silu(_dot(y, w1b[...]) + b1b[...])
    x2 = y + x1
    x3 = _silu(_dot(x2, wm[...]) + bm[...])
    x4 = x3 + h_ref[...]
    y = _silu(_dot(x4, w2a[...]) + b2a[...])
    y = _silu(_dot(y, w2b[...]) + b2b[...])
    x5 = y + x4
    y = _silu(_dot(x5, w3a[...]) + b3a[...])
    y = _silu(_dot(y, w3b[...]) + b3b[...])
    x6 = y + x5
    x6_ref[...] = x6
    p1_ref[...] = _dot(x6, wxe1[...])
    p2_ref[...] = _dot(x6, wxe2[...])


def _mid(part, cnt16, x0, h, weights, block=1000):
    grid = (_N // block,)
    mat = pl.BlockSpec((_D, _D), lambda i: (0, 0))
    vec = pl.BlockSpec((1, _D), lambda i: (0, 0))
    row = pl.BlockSpec((block, _D), lambda i: (i, 0))
    w_specs = []
    for w in weights:
        w_specs.append(mat if w.shape[0] == _D else vec)
    return pl.pallas_call(
        _mid_body,
        grid=grid,
        in_specs=[
            pl.BlockSpec((_NC, block, _D), lambda i: (0, i, 0)),
            pl.BlockSpec((_NC, block, _D), lambda i: (0, i, 0)),
            row, row,
        ] + w_specs,
        out_specs=[row, row, row],
        out_shape=[jax.ShapeDtypeStruct((_N, _D), _F32)] * 3,
    )(part, cnt16, x0, h, *weights)


def _final_body(s_ref, c_ref, x6_ref, out_ref):
    cnt = jnp.maximum(c_ref[0, :, :1] + c_ref[1, :, :1], 1.0)
    out_ref[...] = (s_ref[0] + s_ref[1]) / cnt + x6_ref[...]


def _final(part, cnt16, x6, block=1000):
    grid = (_N // block,)
    row = pl.BlockSpec((block, _D), lambda i: (i, 0))
    return pl.pallas_call(
        _final_body,
        grid=grid,
        in_specs=[
            pl.BlockSpec((_NC, block, _D), lambda i: (0, i, 0)),
            pl.BlockSpec((_NC, block, _D), lambda i: (0, i, 0)),
            row,
        ],
        out_specs=row,
        out_shape=jax.ShapeDtypeStruct((_N, _D), _F32),
    )(part, cnt16, x6)


def _make_prop():
    mesh = plsc.VectorSubcoreMesh(core_axis_name="c", subcore_axis_name="s")
    out_type = jax.ShapeDtypeStruct((_NC, _NP, _D), _F32)
    scratch = [
        pltpu.VMEM((_K,), jnp.int32),
        pltpu.VMEM((_K,), jnp.int32),
        pltpu.VMEM((_K, _D), _F32),
        pltpu.VMEM((_K, _D), _F32),
        pltpu.VMEM((_K, _D), _F32),
        pltpu.VMEM((_K, _D), _F32),
        pltpu.VMEM_SHARED((_NP, _D), _F32),
        pltpu.SemaphoreType.DMA,
        pltpu.SemaphoreType.DMA,
    ]

    def body(idx0_h, idx1_h, t3_h, tl_h, p1_h, p2_h, zrow_h, out_h,
             iv0, iv1, t3v, tlv, p1v, p2v, acc, sem1, sem2):
        c = lax.axis_index("c")
        s = lax.axis_index("s")
        wid = c * _NS + s

        pltpu.sync_copy(zrow_h, acc.at[pl.ds(s * _RPT, _RPT)])
        plsc.subcore_barrier()

        def chunk(g, carry):
            base = wid * _EPW + g * _K
            pltpu.sync_copy(idx0_h.at[pl.ds(base, _K)], iv0)
            pltpu.sync_copy(idx1_h.at[pl.ds(base, _K)], iv1)
            cp1 = pltpu.async_copy(p1_h.at[iv0], p1v, sem1)
            cp2 = pltpu.async_copy(p2_h.at[iv1], p2v, sem2)
            pltpu.sync_copy(t3_h.at[pl.ds(base, _K)], t3v)
            pltpu.sync_copy(tl_h.at[pl.ds(base, _K)], tlv)
            cp1.wait()
            cp2.wait()

            def row(r, rc):
                for v in range(_D // 16):
                    sl = pl.ds(v * 16, 16)
                    y = t3v[r, sl] + p1v[r, sl] + p2v[r, sl]
                    m = tlv[r, sl] * y / (1.0 + jnp.exp(-y))
                    t3v[r, sl] = m
                return rc

            lax.fori_loop(0, _K, row, 0)
            pltpu.sync_copy(t3v, acc.at[iv0], add=True)
            return carry

        lax.fori_loop(0, _NCHUNK, chunk, 0)
        plsc.subcore_barrier()

        sl_out = pl.ds(s * _RPT, _RPT)
        pltpu.sync_copy(acc.at[sl_out], out_h.at[c, sl_out])

    return pl.kernel(body, out_type=out_type, mesh=mesh, scratch_types=scratch)


def _make_count():
    mesh = plsc.VectorSubcoreMesh(core_axis_name="c", subcore_axis_name="s")
    out_type = jax.ShapeDtypeStruct((_NC, _NP, _D), _F32)
    scratch = [
        pltpu.VMEM((_K,), jnp.int32),
        pltpu.VMEM((_K, _D), _F32),
        pltpu.VMEM_SHARED((_NP, _D), _F32),
    ]

    def body(idx0_h, zcnt_h, ones_h, cnt_h, iv0, onesv, cacc):
        c = lax.axis_index("c")
        s = lax.axis_index("s")
        wid = c * _NS + s

        pltpu.sync_copy(zcnt_h, cacc.at[pl.ds(s * _RPT, _RPT)])
        pltpu.sync_copy(ones_h, onesv)
        plsc.subcore_barrier()

        def chunk(g, carry):
            base = wid * _EPW + g * _K
            pltpu.sync_copy(idx0_h.at[pl.ds(base, _K)], iv0)
            pltpu.sync_copy(onesv, cacc.at[iv0], add=True)
            return carry

        lax.fori_loop(0, _NCHUNK, chunk, 0)
        plsc.subcore_barrier()

        sl_out = pl.ds(s * _RPT, _RPT)
        pltpu.sync_copy(cacc.at[sl_out], cnt_h.at[c, sl_out])

    return pl.kernel(body, out_type=out_type, mesh=mesh, scratch_types=scratch)


_prop = _make_prop()
_count = _make_count()


def kernel(h, edge_attr, edge_index, W_h, b_h, W_r1a, b_r1a, W_r1b, b_r1b,
           W_r2a, b_r2a, W_r2b, b_r2b, W_r3a, b_r3a, W_r3b, b_r3b,
           W_m, b_m, W_xe, b_xe, W_lin):
    w1 = W_xe[:_D]
    w2 = W_xe[_D:2 * _D]
    w3 = W_xe[2 * _D:]

    idx0 = edge_index[0]
    idx1 = edge_index[1]

    zrow = jnp.zeros((_RPT, _D), _F32)
    ones = jnp.ones((_K, _D), _F32)

    t3, tl = _edge_proj(edge_attr, w3, b_xe, W_lin)
    x0, p1, p2 = _pre(h, W_h, b_h, w1, w2)

    cnt16 = _count(idx0, zrow, ones)
    part1 = _prop(idx0, idx1, t3, tl, p1, p2, zrow)

    weights = [W_r1a, b_r1a.reshape(1, _D), W_r1b, b_r1b.reshape(1, _D),
               W_m, b_m.reshape(1, _D),
               W_r2a, b_r2a.reshape(1, _D), W_r2b, b_r2b.reshape(1, _D),
               W_r3a, b_r3a.reshape(1, _D), W_r3b, b_r3b.reshape(1, _D),
               w1, w2]
    x6, p1b, p2b = _mid(part1, cnt16, x0, h, weights)

    part2 = _prop(idx0, idx1, t3, tl, p1b, p2b, zrow)

    return _final(part2, cnt16, x6)


# speedup vs baseline: 4.1403x; 1.2730x over previous
import jax
import jax.numpy as jnp
from jax import lax
from jax.experimental import pallas as pl
from jax.experimental.pallas import tpu as pltpu
from jax.experimental.pallas import tpu_sc as plsc

_N = 10000
_E = 320000
_D = 128

_NC = 2
_NS = 16
_NW = _NC * _NS
_EPW = _E // _NW
_K = 80
_NCHUNK = _EPW // _K
_NP = 10240
_RPT = _NP // _NS

_F32 = jnp.float32


def _silu(y):
    return y * jax.nn.sigmoid(y)


def _dot(a, w):
    return jnp.dot(a, w, preferred_element_type=_F32)


def _edge_proj_body(ea_ref, w3_ref, b_ref, wl_ref, t3_ref, tl_ref):
    ea = ea_ref[...]
    t3_ref[...] = _dot(ea, w3_ref[...]) + b_ref[...]
    tl_ref[...] = _dot(ea, wl_ref[...])


def _edge_proj(ea, w3, b_xe, w_lin, block=2000):
    grid = (_E // block,)
    return pl.pallas_call(
        _edge_proj_body,
        grid=grid,
        in_specs=[
            pl.BlockSpec((block, _D), lambda i: (i, 0)),
            pl.BlockSpec((_D, _D), lambda i: (0, 0)),
            pl.BlockSpec((1, _D), lambda i: (0, 0)),
            pl.BlockSpec((_D, _D), lambda i: (0, 0)),
        ],
        out_specs=[
            pl.BlockSpec((block, _D), lambda i: (i, 0)),
            pl.BlockSpec((block, _D), lambda i: (i, 0)),
        ],
        out_shape=[
            jax.ShapeDtypeStruct((_E, _D), _F32),
            jax.ShapeDtypeStruct((_E, _D), _F32),
        ],
    )(ea, w3, b_xe.reshape(1, _D), w_lin)


def _pre_body(h_ref, wh_ref, bh_ref, w1_ref, w2_ref, x0_ref, p1_ref, p2_ref):
    x0 = _silu(_dot(h_ref[...], wh_ref[...]) + bh_ref[...])
    x0_ref[...] = x0
    p1_ref[...] = _dot(x0, w1_ref[...])
    p2_ref[...] = _dot(x0, w2_ref[...])


def _pre(h, w_h, b_h, w1, w2, block=1000):
    grid = (_N // block,)
    mat = pl.BlockSpec((_D, _D), lambda i: (0, 0))
    row = pl.BlockSpec((block, _D), lambda i: (i, 0))
    return pl.pallas_call(
        _pre_body,
        grid=grid,
        in_specs=[row, mat, pl.BlockSpec((1, _D), lambda i: (0, 0)), mat, mat],
        out_specs=[row, row, row],
        out_shape=[jax.ShapeDtypeStruct((_N, _D), _F32)] * 3,
    )(h, w_h, b_h.reshape(1, _D), w1, w2)


def _mid_body(s_ref, c_ref, x0_ref, h_ref,
              w1a, b1a, w1b, b1b, wm, bm,
              w2a, b2a, w2b, b2b, w3a, b3a, w3b, b3b,
              wxe1, wxe2,
              x6_ref, p1_ref, p2_ref):
    cnt = jnp.maximum(c_ref[0, :, :1] + c_ref[1, :, :1], 1.0)
    pooled = (s_ref[0] + s_ref[1]) / cnt
    x1 = pooled + x0_ref[...]
    y = _silu(_dot(x1, w1a[...]) + b1a[...])
    y = _silu(_dot(y, w1b[...]) + b1b[...])
    x2 = y + x1
    x3 = _silu(_dot(x2, wm[...]) + bm[...])
    x4 = x3 + h_ref[...]
    y = _silu(_dot(x4, w2a[...]) + b2a[...])
    y = _silu(_dot(y, w2b[...]) + b2b[...])
    x5 = y + x4
    y = _silu(_dot(x5, w3a[...]) + b3a[...])
    y = _silu(_dot(y, w3b[...]) + b3b[...])
    x6 = y + x5
    x6_ref[...] = x6
    p1_ref[...] = _dot(x6, wxe1[...])
    p2_ref[...] = _dot(x6, wxe2[...])


def _mid(part, cnt16, x0, h, weights, block=1000):
    grid = (_N // block,)
    mat = pl.BlockSpec((_D, _D), lambda i: (0, 0))
    vec = pl.BlockSpec((1, _D), lambda i: (0, 0))
    row = pl.BlockSpec((block, _D), lambda i: (i, 0))
    w_specs = []
    for w in weights:
        w_specs.append(mat if w.shape[0] == _D else vec)
    return pl.pallas_call(
        _mid_body,
        grid=grid,
        in_specs=[
            pl.BlockSpec((_NC, block, _D), lambda i: (0, i, 0)),
            pl.BlockSpec((_NC, block, _D), lambda i: (0, i, 0)),
            row, row,
        ] + w_specs,
        out_specs=[row, row, row],
        out_shape=[jax.ShapeDtypeStruct((_N, _D), _F32)] * 3,
    )(part, cnt16, x0, h, *weights)


def _final_body(s_ref, c_ref, x6_ref, out_ref):
    cnt = jnp.maximum(c_ref[0, :, :1] + c_ref[1, :, :1], 1.0)
    out_ref[...] = (s_ref[0] + s_ref[1]) / cnt + x6_ref[...]


def _final(part, cnt16, x6, block=1000):
    grid = (_N // block,)
    row = pl.BlockSpec((block, _D), lambda i: (i, 0))
    return pl.pallas_call(
        _final_body,
        grid=grid,
        in_specs=[
            pl.BlockSpec((_NC, block, _D), lambda i: (0, i, 0)),
            pl.BlockSpec((_NC, block, _D), lambda i: (0, i, 0)),
            row,
        ],
        out_specs=row,
        out_shape=jax.ShapeDtypeStruct((_N, _D), _F32),
    )(part, cnt16, x6)


def _make_prop():
    mesh = plsc.VectorSubcoreMesh(core_axis_name="c", subcore_axis_name="s")
    out_type = jax.ShapeDtypeStruct((_NC, _NP, _D), _F32)
    scratch = [
        pltpu.VMEM((_K,), jnp.int32),
        pltpu.VMEM((_K,), jnp.int32),
        pltpu.VMEM((_K, _D), _F32),
        pltpu.VMEM((_K, _D), _F32),
        pltpu.VMEM((_K, _D), _F32),
        pltpu.VMEM((_K, _D), _F32),
        pltpu.VMEM_SHARED((_NP, _D), _F32),
        [pltpu.SemaphoreType.DMA] * 2,
        [pltpu.SemaphoreType.DMA] * 2,
        [pltpu.SemaphoreType.DMA] * 2,
    ]

    def body(idx0_h, idx1_h, t3_h, tl_h, p1_h, p2_h, zrow_h, out_h,
             iv0, iv1, t3v, tlv, p1v, p2v, acc, isem, lsem, gsem):
        c = lax.axis_index("c")
        s = lax.axis_index("s")
        wid = c * _NS + s

        pltpu.sync_copy(zrow_h, acc.at[pl.ds(s * _RPT, _RPT)])
        plsc.subcore_barrier()

        def chunk(g, carry):
            base = wid * _EPW + g * _K
            sl = pl.ds(base, _K)
            ci0 = pltpu.async_copy(idx0_h.at[sl], iv0, isem[0])
            ci1 = pltpu.async_copy(idx1_h.at[sl], iv1, isem[1])
            ct3 = pltpu.async_copy(t3_h.at[sl], t3v, lsem[0])
            ctl = pltpu.async_copy(tl_h.at[sl], tlv, lsem[1])
            ci0.wait()
            ci1.wait()
            cp1 = pltpu.async_copy(p1_h.at[iv0], p1v, gsem[0])
            cp2 = pltpu.async_copy(p2_h.at[iv1], p2v, gsem[1])
            ct3.wait()
            ctl.wait()
            cp1.wait()
            cp2.wait()

            def row(r, rc):
                for v in range(_D // 16):
                    rsl = pl.ds(v * 16, 16)
                    y = t3v[r, rsl] + p1v[r, rsl] + p2v[r, rsl]
                    m = tlv[r, rsl] * y / (1.0 + jnp.exp(-y))
                    t3v[r, rsl] = m
                return rc

            lax.fori_loop(0, _K, row, 0)
            pltpu.sync_copy(t3v, acc.at[iv0], add=True)
            return carry

        lax.fori_loop(0, _NCHUNK, chunk, 0)
        plsc.subcore_barrier()

        sl_out = pl.ds(s * _RPT, _RPT)
        pltpu.sync_copy(acc.at[sl_out], out_h.at[c, sl_out])

    return pl.kernel(body, out_type=out_type, mesh=mesh, scratch_types=scratch)


def _make_count():
    mesh = plsc.VectorSubcoreMesh(core_axis_name="c", subcore_axis_name="s")
    out_type = jax.ShapeDtypeStruct((_NC, _NP, _D), _F32)
    scratch = [
        pltpu.VMEM((_NCHUNK, _K), jnp.int32),
        pltpu.VMEM((_K, _D), _F32),
        pltpu.VMEM_SHARED((_NP, _D), _F32),
        pltpu.SemaphoreType.DMA,
    ]

    def body(idx0_h, zcnt_h, ones_h, cnt_h, iva, onesv, cacc, sem):
        c = lax.axis_index("c")
        s = lax.axis_index("s")
        wid = c * _NS + s

        pltpu.sync_copy(zcnt_h, cacc.at[pl.ds(s * _RPT, _RPT)])
        pltpu.sync_copy(ones_h, onesv)
        pltpu.sync_copy(idx0_h.at[wid], iva)
        plsc.subcore_barrier()

        nb = _NCHUNK // 5

        def issue(t):
            for j in range(5):
                pltpu.async_copy(onesv, cacc.at[iva.at[5 * t + j]], sem,
                                 add=True)

        def drain(t):
            for j in range(5):
                pltpu.make_async_copy(onesv, cacc.at[iva.at[5 * t + j]],
                                      sem).wait()

        issue(0)

        def batch(t, carry):
            issue(t + 1)
            drain(t)
            return carry

        lax.fori_loop(0, nb - 1, batch, 0)
        drain(nb - 1)
        plsc.subcore_barrier()

        sl_out = pl.ds(s * _RPT, _RPT)
        pltpu.sync_copy(cacc.at[sl_out], cnt_h.at[c, sl_out])

    return pl.kernel(body, out_type=out_type, mesh=mesh, scratch_types=scratch)


_prop = _make_prop()
_count = _make_count()


def kernel(h, edge_attr, edge_index, W_h, b_h, W_r1a, b_r1a, W_r1b, b_r1b,
           W_r2a, b_r2a, W_r2b, b_r2b, W_r3a, b_r3a, W_r3b, b_r3b,
           W_m, b_m, W_xe, b_xe, W_lin):
    w1 = W_xe[:_D]
    w2 = W_xe[_D:2 * _D]
    w3 = W_xe[2 * _D:]

    idx0 = edge_index[0]
    idx1 = edge_index[1]
    idx0r = idx0.reshape(_NW, _NCHUNK, _K)

    zrow = jnp.zeros((_RPT, _D), _F32)
    ones = jnp.ones((_K, _D), _F32)

    t3, tl = _edge_proj(edge_attr, w3, b_xe, W_lin)
    x0, p1, p2 = _pre(h, W_h, b_h, w1, w2)

    cnt16 = _count(idx0r, zrow, ones)
    part1 = _prop(idx0, idx1, t3, tl, p1, p2, zrow)

    weights = [W_r1a, b_r1a.reshape(1, _D), W_r1b, b_r1b.reshape(1, _D),
               W_m, b_m.reshape(1, _D),
               W_r2a, b_r2a.reshape(1, _D), W_r2b, b_r2b.reshape(1, _D),
               W_r3a, b_r3a.reshape(1, _D), W_r3b, b_r3b.reshape(1, _D),
               w1, w2]
    x6, p1b, p2b = _mid(part1, cnt16, x0, h, weights)

    part2 = _prop(idx0, idx1, t3, tl, p1b, p2b, zrow)

    return _final(part2, cnt16, x6)


# speedup vs baseline: 4.3609x; 1.0533x over previous
import jax
import jax.numpy as jnp
from jax import lax
from jax.experimental import pallas as pl
from jax.experimental.pallas import tpu as pltpu
from jax.experimental.pallas import tpu_sc as plsc

_N = 10000
_E = 320000
_D = 128

_NC = 2
_NS = 16
_NW = _NC * _NS
_EPW = _E // _NW
_K = 80
_NCHUNK = _EPW // _K
_NP = 10240
_RPT = _NP // _NS

_F32 = jnp.float32


def _silu(y):
    return y * jax.nn.sigmoid(y)


def _dot(a, w):
    return jnp.dot(a, w, preferred_element_type=_F32)


def _edge_proj_body(ea_ref, w3_ref, b_ref, wl_ref, t3_ref, tl_ref):
    ea = ea_ref[...]
    t3_ref[...] = _dot(ea, w3_ref[...]) + b_ref[...]
    tl_ref[...] = _dot(ea, wl_ref[...])


def _edge_proj(ea, w3, b_xe, w_lin, block=2000):
    grid = (_E // block,)
    return pl.pallas_call(
        _edge_proj_body,
        grid=grid,
        in_specs=[
            pl.BlockSpec((block, _D), lambda i: (i, 0)),
            pl.BlockSpec((_D, _D), lambda i: (0, 0)),
            pl.BlockSpec((1, _D), lambda i: (0, 0)),
            pl.BlockSpec((_D, _D), lambda i: (0, 0)),
        ],
        out_specs=[
            pl.BlockSpec((block, _D), lambda i: (i, 0)),
            pl.BlockSpec((block, _D), lambda i: (i, 0)),
        ],
        out_shape=[
            jax.ShapeDtypeStruct((_E, _D), _F32),
            jax.ShapeDtypeStruct((_E, _D), _F32),
        ],
    )(ea, w3, b_xe.reshape(1, _D), w_lin)


def _pre_body(h_ref, wh_ref, bh_ref, w1_ref, w2_ref, x0_ref, p1_ref, p2_ref):
    x0 = _silu(_dot(h_ref[...], wh_ref[...]) + bh_ref[...])
    x0_ref[...] = x0
    p1_ref[...] = _dot(x0, w1_ref[...])
    p2_ref[...] = _dot(x0, w2_ref[...])


def _pre(h, w_h, b_h, w1, w2, block=1000):
    grid = (_N // block,)
    mat = pl.BlockSpec((_D, _D), lambda i: (0, 0))
    row = pl.BlockSpec((block, _D), lambda i: (i, 0))
    return pl.pallas_call(
        _pre_body,
        grid=grid,
        in_specs=[row, mat, pl.BlockSpec((1, _D), lambda i: (0, 0)), mat, mat],
        out_specs=[row, row, row],
        out_shape=[jax.ShapeDtypeStruct((_N, _D), _F32)] * 3,
    )(h, w_h, b_h.reshape(1, _D), w1, w2)


def _mid_body(s_ref, c_ref, x0_ref, h_ref,
              w1a, b1a, w1b, b1b, wm, bm,
              w2a, b2a, w2b, b2b, w3a, b3a, w3b, b3b,
              wxe1, wxe2,
              x6_ref, p1_ref, p2_ref):
    cnt = jnp.maximum(c_ref[0, :, :1] + c_ref[1, :, :1], 1.0)
    pooled = (s_ref[0] + s_ref[1]) / cnt
    x1 = pooled + x0_ref[...]
    y = _silu(_dot(x1, w1a[...]) + b1a[...])
    y = _silu(_dot(y, w1b[...]) + b1b[...])
    x2 = y + x1
    x3 = _silu(_dot(x2, wm[...]) + bm[...])
    x4 = x3 + h_ref[...]
    y = _silu(_dot(x4, w2a[...]) + b2a[...])
    y = _silu(_dot(y, w2b[...]) + b2b[...])
    x5 = y + x4
    y = _silu(_dot(x5, w3a[...]) + b3a[...])
    y = _silu(_dot(y, w3b[...]) + b3b[...])
    x6 = y + x5
    x6_ref[...] = x6
    p1_ref[...] = _dot(x6, wxe1[...])
    p2_ref[...] = _dot(x6, wxe2[...])


def _mid(part, cnt16, x0, h, weights, block=1000):
    grid = (_N // block,)
    mat = pl.BlockSpec((_D, _D), lambda i: (0, 0))
    vec = pl.BlockSpec((1, _D), lambda i: (0, 0))
    row = pl.BlockSpec((block, _D), lambda i: (i, 0))
    w_specs = []
    for w in weights:
        w_specs.append(mat if w.shape[0] == _D else vec)
    return pl.pallas_call(
        _mid_body,
        grid=grid,
        in_specs=[
            pl.BlockSpec((_NC, block, _D), lambda i: (0, i, 0)),
            pl.BlockSpec((_NC, block, _D), lambda i: (0, i, 0)),
            row, row,
        ] + w_specs,
        out_specs=[row, row, row],
        out_shape=[jax.ShapeDtypeStruct((_N, _D), _F32)] * 3,
    )(part, cnt16, x0, h, *weights)


def _final_body(s_ref, c_ref, x6_ref, out_ref):
    cnt = jnp.maximum(c_ref[0, :, :1] + c_ref[1, :, :1], 1.0)
    out_ref[...] = (s_ref[0] + s_ref[1]) / cnt + x6_ref[...]


def _final(part, cnt16, x6, block=1000):
    grid = (_N // block,)
    row = pl.BlockSpec((block, _D), lambda i: (i, 0))
    return pl.pallas_call(
        _final_body,
        grid=grid,
        in_specs=[
            pl.BlockSpec((_NC, block, _D), lambda i: (0, i, 0)),
            pl.BlockSpec((_NC, block, _D), lambda i: (0, i, 0)),
            row,
        ],
        out_specs=row,
        out_shape=jax.ShapeDtypeStruct((_N, _D), _F32),
    )(part, cnt16, x6)


def _make_prop():
    mesh = plsc.VectorSubcoreMesh(core_axis_name="c", subcore_axis_name="s")
    out_type = jax.ShapeDtypeStruct((_NC, _NP, _D), _F32)
    scratch = [
        pltpu.VMEM((_K,), jnp.int32),
        pltpu.VMEM((_K,), jnp.int32),
        pltpu.VMEM((_K, _D), _F32),
        pltpu.VMEM((_K, _D), _F32),
        pltpu.VMEM((_K, _D), _F32),
        pltpu.VMEM((_K, _D), _F32),
        pltpu.VMEM_SHARED((_NP, _D), _F32),
        [pltpu.SemaphoreType.DMA] * 2,
        [pltpu.SemaphoreType.DMA] * 2,
        [pltpu.SemaphoreType.DMA] * 2,
    ]

    def body(idx0_h, idx1_h, t3_h, tl_h, p1_h, p2_h, zrow_h, out_h,
             iv0, iv1, t3v, tlv, p1v, p2v, acc, isem, lsem, gsem):
        c = lax.axis_index("c")
        s = lax.axis_index("s")
        wid = c * _NS + s

        pltpu.sync_copy(zrow_h, acc.at[pl.ds(s * _RPT, _RPT)])
        plsc.subcore_barrier()

        def chunk(g, carry):
            base = wid * _EPW + g * _K
            sl = pl.ds(base, _K)
            ci0 = pltpu.async_copy(idx0_h.at[sl], iv0, isem[0])
            ci1 = pltpu.async_copy(idx1_h.at[sl], iv1, isem[1])
            ct3 = pltpu.async_copy(t3_h.at[sl], t3v, lsem[0])
            ctl = pltpu.async_copy(tl_h.at[sl], tlv, lsem[1])
            ci0.wait()
            ci1.wait()
            hh = _K // 2
            slA, slB = pl.ds(0, hh), pl.ds(hh, hh)
            cp1 = pltpu.async_copy(p1_h.at[iv0.at[slA]], p1v.at[slA], gsem[0])
            cp2 = pltpu.async_copy(p2_h.at[iv1.at[slA]], p2v.at[slA], gsem[1])
            ct3.wait()
            ctl.wait()
            cp1.wait()
            cp2.wait()
            cp3 = pltpu.async_copy(p1_h.at[iv0.at[slB]], p1v.at[slB], gsem[0])
            cp4 = pltpu.async_copy(p2_h.at[iv1.at[slB]], p2v.at[slB], gsem[1])

            def row(r, rc):
                for v in range(_D // 16):
                    rsl = pl.ds(v * 16, 16)
                    y = t3v[r, rsl] + p1v[r, rsl] + p2v[r, rsl]
                    m = tlv[r, rsl] * y / (1.0 + jnp.exp(-y))
                    t3v[r, rsl] = m
                return rc

            lax.fori_loop(0, hh, row, 0)
            cp3.wait()
            cp4.wait()
            lax.fori_loop(hh, _K, row, 0)
            pltpu.sync_copy(t3v, acc.at[iv0], add=True)
            return carry

        lax.fori_loop(0, _NCHUNK, chunk, 0)
        plsc.subcore_barrier()

        sl_out = pl.ds(s * _RPT, _RPT)
        pltpu.sync_copy(acc.at[sl_out], out_h.at[c, sl_out])

    return pl.kernel(body, out_type=out_type, mesh=mesh, scratch_types=scratch)


def _make_count():
    mesh = plsc.VectorSubcoreMesh(core_axis_name="c", subcore_axis_name="s")
    out_type = jax.ShapeDtypeStruct((_NC, _NP, _D), _F32)
    scratch = [
        pltpu.VMEM((_NCHUNK, _K), jnp.int32),
        pltpu.VMEM((_K, _D), _F32),
        pltpu.VMEM_SHARED((_NP, _D), _F32),
        pltpu.SemaphoreType.DMA,
    ]

    def body(idx0_h, zcnt_h, ones_h, cnt_h, iva, onesv, cacc, sem):
        c = lax.axis_index("c")
        s = lax.axis_index("s")
        wid = c * _NS + s

        pltpu.sync_copy(zcnt_h, cacc.at[pl.ds(s * _RPT, _RPT)])
        pltpu.sync_copy(ones_h, onesv)
        pltpu.sync_copy(idx0_h.at[wid], iva)
        plsc.subcore_barrier()

        nb = _NCHUNK // 5

        def issue(t):
            for j in range(5):
                pltpu.async_copy(onesv, cacc.at[iva.at[5 * t + j]], sem,
                                 add=True)

        def drain(t):
            for j in range(5):
                pltpu.make_async_copy(onesv, cacc.at[iva.at[5 * t + j]],
                                      sem).wait()

        issue(0)

        def batch(t, carry):
            issue(t + 1)
            drain(t)
            return carry

        lax.fori_loop(0, nb - 1, batch, 0)
        drain(nb - 1)
        plsc.subcore_barrier()

        sl_out = pl.ds(s * _RPT, _RPT)
        pltpu.sync_copy(cacc.at[sl_out], cnt_h.at[c, sl_out])

    return pl.kernel(body, out_type=out_type, mesh=mesh, scratch_types=scratch)


_prop = _make_prop()
_count = _make_count()


def kernel(h, edge_attr, edge_index, W_h, b_h, W_r1a, b_r1a, W_r1b, b_r1b,
           W_r2a, b_r2a, W_r2b, b_r2b, W_r3a, b_r3a, W_r3b, b_r3b,
           W_m, b_m, W_xe, b_xe, W_lin):
    w1 = W_xe[:_D]
    w2 = W_xe[_D:2 * _D]
    w3 = W_xe[2 * _D:]

    idx0 = edge_index[0]
    idx1 = edge_index[1]
    idx0r = idx0.reshape(_NW, _NCHUNK, _K)

    zrow = jnp.zeros((_RPT, _D), _F32)
    ones = jnp.ones((_K, _D), _F32)

    t3, tl = _edge_proj(edge_attr, w3, b_xe, W_lin)
    x0, p1, p2 = _pre(h, W_h, b_h, w1, w2)

    cnt16 = _count(idx0r, zrow, ones)
    part1 = _prop(idx0, idx1, t3, tl, p1, p2, zrow)

    weights = [W_r1a, b_r1a.reshape(1, _D), W_r1b, b_r1b.reshape(1, _D),
               W_m, b_m.reshape(1, _D),
               W_r2a, b_r2a.reshape(1, _D), W_r2b, b_r2b.reshape(1, _D),
               W_r3a, b_r3a.reshape(1, _D), W_r3b, b_r3b.reshape(1, _D),
               w1, w2]
    x6, p1b, p2b = _mid(part1, cnt16, x0, h, weights)

    part2 = _prop(idx0, idx1, t3, tl, p1b, p2b, zrow)

    return _final(part2, cnt16, x6)
